# Initial kernel scaffold; baseline (speedup 1.0000x reference)
import jax
import jax.numpy as jnp
from jax.experimental import pallas as pl


def kernel(x, edge_index, W1, b1, gcn_W1, l1_w, l1_b, W2, b2, gcn_W2, l2_w, l2_b):
    raise NotImplementedError("write your pallas kernel here")


# speedup vs baseline: 11.0523x; 11.0523x over previous
import functools

import jax
import jax.numpy as jnp
from jax import lax
from jax.experimental import pallas as pl
from jax.experimental.pallas import tpu as pltpu
from jax.experimental.pallas import tpu_sc as plsc

N = 10000
E = 320000
D = 128
C = 40
EPS = 0.1
GAMMA = 0.1

NC = 2
NS = 16
NW = NC * NS
EPT = E // NW
KE = 80
NIT = EPT // KE
NRT = 10
RPT = N // NRT

_mesh = plsc.VectorSubcoreMesh(core_axis_name="c", subcore_axis_name="s")


@functools.partial(
    pl.kernel,
    out_type=jax.ShapeDtypeStruct((NC, N, D), jnp.float32),
    mesh=_mesh,
    scratch_types=[
        pltpu.VMEM((KE,), jnp.int32),
        pltpu.VMEM((KE, D), jnp.float32),
        pltpu.VMEM_SHARED((N, D), jnp.float32),
    ],
)
def _sc_deg(dst_hbm, ones_hbm, zeros_hbm, out_hbm, idx_v, ones_v, acc_sh):
    c = lax.axis_index("c")
    s = lax.axis_index("s")
    wid = c * NS + s

    @pl.when(s < NRT)
    def _init():
        pltpu.sync_copy(zeros_hbm, acc_sh.at[pl.ds(s * RPT, RPT)])

    pltpu.sync_copy(ones_hbm, ones_v)
    plsc.subcore_barrier()

    def body(g, _):
        base = wid * EPT + g * KE
        pltpu.sync_copy(dst_hbm.at[pl.ds(base, KE)], idx_v)
        pltpu.sync_copy(ones_v, acc_sh.at[idx_v], add=True)
        return _

    lax.fori_loop(0, NIT, body, 0)
    plsc.subcore_barrier()

    @pl.when(s < NRT)
    def _readback():
        pltpu.sync_copy(
            acc_sh.at[pl.ds(s * RPT, RPT)], out_hbm.at[c, pl.ds(s * RPT, RPT)]
        )


@functools.partial(
    pl.kernel,
    out_type=jax.ShapeDtypeStruct((NC, N, D), jnp.float32),
    mesh=_mesh,
    scratch_types=[
        pltpu.VMEM((KE,), jnp.int32),
        pltpu.VMEM((KE,), jnp.int32),
        pltpu.VMEM((KE, D), jnp.float32),
        pltpu.VMEM_SHARED((N, D), jnp.float32),
        pltpu.SemaphoreType.DMA,
    ],
)
def _sc_scatter(hp_hbm, src_hbm, dst_hbm, zeros_hbm, out_hbm,
                src_v, dst_v, rows_v, acc_sh, sem):
    c = lax.axis_index("c")
    s = lax.axis_index("s")
    wid = c * NS + s

    @pl.when(s < NRT)
    def _init():
        pltpu.sync_copy(zeros_hbm, acc_sh.at[pl.ds(s * RPT, RPT)])

    plsc.subcore_barrier()

    def body(g, _):
        base = wid * EPT + g * KE
        pltpu.sync_copy(src_hbm.at[pl.ds(base, KE)], src_v)
        pltpu.sync_copy(dst_hbm.at[pl.ds(base, KE)], dst_v)
        pltpu.async_copy(hp_hbm.at[src_v], rows_v, sem).wait()
        pltpu.sync_copy(rows_v, acc_sh.at[dst_v], add=True)
        return _

    lax.fori_loop(0, NIT, body, 0)
    plsc.subcore_barrier()

    @pl.when(s < NRT)
    def _readback():
        pltpu.sync_copy(
            acc_sh.at[pl.ds(s * RPT, RPT)], out_hbm.at[c, pl.ds(s * RPT, RPT)]
        )


BN = 2000
GRID = N // BN

_HI = lax.Precision.HIGHEST


def _mm(a, b, ca, cb):
    return lax.dot_general(
        a, b, (((ca,), (cb,)), ((), ())),
        precision=_HI, preferred_element_type=jnp.float32)


def _t1_body(deg_ref, x_ref, w_ref, dinv_ref, hp_ref):
    deg = deg_ref[0, :, 0:1] + deg_ref[1, :, 0:1] + 1.0
    dinv = lax.rsqrt(jnp.maximum(deg, 1e-12))
    dinv_ref[...] = dinv
    hp_ref[...] = _mm(x_ref[...], w_ref[...], 1, 0) * dinv


_t1 = pl.pallas_call(
    _t1_body,
    grid=(GRID,),
    in_specs=[
        pl.BlockSpec((NC, BN, D), lambda i: (0, i, 0)),
        pl.BlockSpec((BN, D), lambda i: (i, 0)),
        pl.BlockSpec((D, D), lambda i: (0, 0)),
    ],
    out_specs=[
        pl.BlockSpec((BN, 1), lambda i: (i, 0)),
        pl.BlockSpec((BN, D), lambda i: (i, 0)),
    ],
    out_shape=[
        jax.ShapeDtypeStruct((N, 1), jnp.float32),
        jax.ShapeDtypeStruct((N, D), jnp.float32),
    ],
)


def _t2_body(x_ref, acc_ref, hp_ref, dinv_ref, w1_ref, b1_ref,
             l1w_ref, l1b_ref, gw2_ref, y_ref, h2p_ref):
    x = x_ref[...]
    dinv = dinv_ref[...]
    gcn = (acc_ref[0] + acc_ref[1] + hp_ref[...]) * dinv
    lin = _mm(x, w1_ref[...], 1, 1) - _mm(x, w1_ref[...], 1, 0) - GAMMA * x
    z = lin + gcn + b1_ref[...]
    x1 = jnp.maximum(x + EPS * jnp.tanh(z), 0.0)
    y = jnp.maximum(_mm(x1, l1w_ref[...], 1, 1) + l1b_ref[...], 0.0)
    y_ref[...] = y
    h2p_ref[...] = _mm(y, gw2_ref[...], 1, 0) * dinv


_t2 = pl.pallas_call(
    _t2_body,
    grid=(GRID,),
    in_specs=[
        pl.BlockSpec((BN, D), lambda i: (i, 0)),
        pl.BlockSpec((NC, BN, D), lambda i: (0, i, 0)),
        pl.BlockSpec((BN, D), lambda i: (i, 0)),
        pl.BlockSpec((BN, 1), lambda i: (i, 0)),
        pl.BlockSpec((D, D), lambda i: (0, 0)),
        pl.BlockSpec((1, D), lambda i: (0, 0)),
        pl.BlockSpec((D, D), lambda i: (0, 0)),
        pl.BlockSpec((1, D), lambda i: (0, 0)),
        pl.BlockSpec((D, D), lambda i: (0, 0)),
    ],
    out_specs=[
        pl.BlockSpec((BN, D), lambda i: (i, 0)),
        pl.BlockSpec((BN, D), lambda i: (i, 0)),
    ],
    out_shape=[
        jax.ShapeDtypeStruct((N, D), jnp.float32),
        jax.ShapeDtypeStruct((N, D), jnp.float32),
    ],
)


def _t3_body(y_ref, acc_ref, hp_ref, dinv_ref, w2_ref, b2_ref,
             l2w_ref, l2b_ref, res_ref, x2_ref):
    y = y_ref[...]
    dinv = dinv_ref[...]
    gcn = (acc_ref[0] + acc_ref[1] + hp_ref[...]) * dinv
    lin = _mm(y, w2_ref[...], 1, 1) - _mm(y, w2_ref[...], 1, 0) - GAMMA * y
    z = lin + gcn + b2_ref[...]
    x2 = jnp.maximum(y + EPS * jnp.tanh(z), 0.0)
    x2_ref[...] = x2
    logits = _mm(x2, l2w_ref[...], 1, 1) + l2b_ref[...]
    m = jnp.max(logits, axis=-1, keepdims=True)
    lse = m + jnp.log(jnp.sum(jnp.exp(logits - m), axis=-1, keepdims=True))
    res_ref[...] = logits - lse


_t3 = pl.pallas_call(
    _t3_body,
    grid=(GRID,),
    in_specs=[
        pl.BlockSpec((BN, D), lambda i: (i, 0)),
        pl.BlockSpec((NC, BN, D), lambda i: (0, i, 0)),
        pl.BlockSpec((BN, D), lambda i: (i, 0)),
        pl.BlockSpec((BN, 1), lambda i: (i, 0)),
        pl.BlockSpec((D, D), lambda i: (0, 0)),
        pl.BlockSpec((1, D), lambda i: (0, 0)),
        pl.BlockSpec((C, D), lambda i: (0, 0)),
        pl.BlockSpec((1, C), lambda i: (0, 0)),
    ],
    out_specs=[
        pl.BlockSpec((BN, C), lambda i: (i, 0)),
        pl.BlockSpec((BN, D), lambda i: (i, 0)),
    ],
    out_shape=[
        jax.ShapeDtypeStruct((N, C), jnp.float32),
        jax.ShapeDtypeStruct((N, D), jnp.float32),
    ],
)


def kernel(x, edge_index, W1, b1, gcn_W1, l1_w, l1_b, W2, b2, gcn_W2,
           l2_w, l2_b):
    src = edge_index[0]
    dst = edge_index[1]
    zeros_rows = jnp.zeros((RPT, D), jnp.float32)
    ones_rows = jnp.ones((KE, D), jnp.float32)

    deg_parts = _sc_deg(dst, ones_rows, zeros_rows)
    dinv, h1p = _t1(deg_parts, x, gcn_W1)
    acc1 = _sc_scatter(h1p, src, dst, zeros_rows)
    y, h2p = _t2(x, acc1, h1p, dinv, W1, b1.reshape(1, D),
                 l1_w, l1_b.reshape(1, D), gcn_W2)
    acc2 = _sc_scatter(h2p, src, dst, zeros_rows)
    res, x2 = _t3(y, acc2, h2p, dinv, W2, b2.reshape(1, D),
                  l2_w, l2_b.reshape(1, C))
    return (res, x2)


# speedup vs baseline: 19.4753x; 1.7621x over previous
import functools

import jax
import jax.numpy as jnp
from jax import lax
from jax.experimental import pallas as pl
from jax.experimental.pallas import tpu as pltpu
from jax.experimental.pallas import tpu_sc as plsc

N = 10000
E = 320000
D = 128
C = 40
EPS = 0.1
GAMMA = 0.1

NC = 2
NS = 16
NW = NC * NS
EPT = E // NW
KE = 80
NIT = EPT // KE
KS = 128
CH = E // KS
NRT = 10
RPT = N // NRT

_mesh = plsc.VectorSubcoreMesh(core_axis_name="c", subcore_axis_name="s")


@functools.partial(
    pl.kernel,
    out_type=jax.ShapeDtypeStruct((NC, N, D), jnp.float32),
    mesh=_mesh,
    scratch_types=[
        pltpu.VMEM((KE,), jnp.int32),
        pltpu.VMEM((KE, D), jnp.float32),
        pltpu.VMEM_SHARED((N, D), jnp.float32),
    ],
)
def _sc_deg(dst_hbm, ones_hbm, zeros_hbm, out_hbm, idx_v, ones_v, acc_sh):
    c = lax.axis_index("c")
    s = lax.axis_index("s")
    wid = c * NS + s

    @pl.when(s < NRT)
    def _init():
        pltpu.sync_copy(zeros_hbm, acc_sh.at[pl.ds(s * RPT, RPT)])

    pltpu.sync_copy(ones_hbm, ones_v)
    plsc.subcore_barrier()

    def body(g, _):
        base = wid * EPT + g * KE
        pltpu.sync_copy(dst_hbm.at[pl.ds(base, KE)], idx_v)
        pltpu.sync_copy(ones_v, acc_sh.at[idx_v], add=True)
        return _

    lax.fori_loop(0, NIT, body, 0)
    plsc.subcore_barrier()

    @pl.when(s < NRT)
    def _readback():
        pltpu.sync_copy(
            acc_sh.at[pl.ds(s * RPT, RPT)], out_hbm.at[c, pl.ds(s * RPT, RPT)]
        )


@functools.partial(
    pl.kernel,
    out_type=jax.ShapeDtypeStruct((NC, N, D), jnp.float32),
    mesh=_mesh,
    scratch_types=[
        pltpu.VMEM((2, KS), jnp.int32),
        pltpu.VMEM((2, KS), jnp.int32),
        pltpu.VMEM((2, KS, D), jnp.float32),
        pltpu.VMEM_SHARED((N, D), jnp.float32),
        pltpu.SemaphoreType.DMA,
        pltpu.SemaphoreType.DMA,
        pltpu.SemaphoreType.DMA,
        pltpu.SemaphoreType.DMA,
    ],
)
def _sc_scatter(hp_hbm, src_hbm, dst_hbm, zeros_hbm, out_hbm,
                sidx, didx, rows_v, acc_sh, sem_si, sem_di, sem_g, sem_s):
    c = lax.axis_index("c")
    s = lax.axis_index("s")
    wid = c * NS + s
    lo = wid * CH // NW
    hi = (wid + 1) * CH // NW

    def fire_sidx(g, b):
        pltpu.async_copy(src_hbm.at[pl.ds(g * KS, KS)], sidx.at[b], sem_si)

    def fire_didx(g, b):
        pltpu.async_copy(dst_hbm.at[pl.ds(g * KS, KS)], didx.at[b], sem_di)

    def drain_idx(b, sem):
        pltpu.make_async_copy(src_hbm.at[pl.ds(0, KS)], sidx.at[b], sem).wait()

    def drain_rows(b, sem):
        pltpu.make_async_copy(hp_hbm.at[pl.ds(0, KS)], rows_v.at[b], sem).wait()

    fire_sidx(lo, 0)
    fire_didx(lo, 0)

    @pl.when(s < NRT)
    def _init():
        pltpu.sync_copy(zeros_hbm, acc_sh.at[pl.ds(s * RPT, RPT)])

    drain_idx(0, sem_si)
    pltpu.async_copy(hp_hbm.at[sidx.at[0]], rows_v.at[0], sem_g)

    @pl.when(lo + 1 < hi)
    def _pre2():
        fire_sidx(lo + 1, 1)

    plsc.subcore_barrier()

    def body(g, carry):
        b = lax.rem(g - lo, 2)
        drain_rows(b, sem_g)

        @pl.when(g > lo)
        def _():
            drain_rows(1 - b, sem_s)

        drain_idx(b, sem_di)
        pltpu.async_copy(rows_v.at[b], acc_sh.at[didx.at[b]], sem_s, add=True)

        @pl.when(g + 1 < hi)
        def _():
            fire_didx(g + 1, 1 - b)
            drain_idx(1 - b, sem_si)
            pltpu.async_copy(hp_hbm.at[sidx.at[1 - b]], rows_v.at[1 - b], sem_g)

        @pl.when(g + 2 < hi)
        def _():
            fire_sidx(g + 2, b)

        return carry

    lax.fori_loop(lo, hi, body, 0)
    bl = lax.rem(hi - 1 - lo, 2)
    drain_rows(bl, sem_s)
    plsc.subcore_barrier()

    @pl.when(s < NRT)
    def _readback():
        pltpu.sync_copy(
            acc_sh.at[pl.ds(s * RPT, RPT)], out_hbm.at[c, pl.ds(s * RPT, RPT)]
        )


BN = 2000
GRID = N // BN

_HI = lax.Precision.HIGHEST


def _mm(a, b, ca, cb):
    return lax.dot_general(
        a, b, (((ca,), (cb,)), ((), ())),
        precision=_HI, preferred_element_type=jnp.float32)


def _t1_body(deg_ref, x_ref, w_ref, dinv_ref, hp_ref):
    deg = deg_ref[0, :, 0:1] + deg_ref[1, :, 0:1] + 1.0
    dinv = lax.rsqrt(jnp.maximum(deg, 1e-12))
    dinv_ref[...] = dinv
    hp_ref[...] = _mm(x_ref[...], w_ref[...], 1, 0) * dinv


_t1 = pl.pallas_call(
    _t1_body,
    grid=(GRID,),
    in_specs=[
        pl.BlockSpec((NC, BN, D), lambda i: (0, i, 0)),
        pl.BlockSpec((BN, D), lambda i: (i, 0)),
        pl.BlockSpec((D, D), lambda i: (0, 0)),
    ],
    out_specs=[
        pl.BlockSpec((BN, 1), lambda i: (i, 0)),
        pl.BlockSpec((BN, D), lambda i: (i, 0)),
    ],
    out_shape=[
        jax.ShapeDtypeStruct((N, 1), jnp.float32),
        jax.ShapeDtypeStruct((N, D), jnp.float32),
    ],
)


def _t2_body(x_ref, acc_ref, hp_ref, dinv_ref, w1_ref, b1_ref,
             l1w_ref, l1b_ref, gw2_ref, y_ref, h2p_ref):
    x = x_ref[...]
    dinv = dinv_ref[...]
    gcn = (acc_ref[0] + acc_ref[1] + hp_ref[...]) * dinv
    lin = _mm(x, w1_ref[...], 1, 1) - _mm(x, w1_ref[...], 1, 0) - GAMMA * x
    z = lin + gcn + b1_ref[...]
    x1 = jnp.maximum(x + EPS * jnp.tanh(z), 0.0)
    y = jnp.maximum(_mm(x1, l1w_ref[...], 1, 1) + l1b_ref[...], 0.0)
    y_ref[...] = y
    h2p_ref[...] = _mm(y, gw2_ref[...], 1, 0) * dinv


_t2 = pl.pallas_call(
    _t2_body,
    grid=(GRID,),
    in_specs=[
        pl.BlockSpec((BN, D), lambda i: (i, 0)),
        pl.BlockSpec((NC, BN, D), lambda i: (0, i, 0)),
        pl.BlockSpec((BN, D), lambda i: (i, 0)),
        pl.BlockSpec((BN, 1), lambda i: (i, 0)),
        pl.BlockSpec((D, D), lambda i: (0, 0)),
        pl.BlockSpec((1, D), lambda i: (0, 0)),
        pl.BlockSpec((D, D), lambda i: (0, 0)),
        pl.BlockSpec((1, D), lambda i: (0, 0)),
        pl.BlockSpec((D, D), lambda i: (0, 0)),
    ],
    out_specs=[
        pl.BlockSpec((BN, D), lambda i: (i, 0)),
        pl.BlockSpec((BN, D), lambda i: (i, 0)),
    ],
    out_shape=[
        jax.ShapeDtypeStruct((N, D), jnp.float32),
        jax.ShapeDtypeStruct((N, D), jnp.float32),
    ],
)


def _t3_body(y_ref, acc_ref, hp_ref, dinv_ref, w2_ref, b2_ref,
             l2w_ref, l2b_ref, res_ref, x2_ref):
    y = y_ref[...]
    dinv = dinv_ref[...]
    gcn = (acc_ref[0] + acc_ref[1] + hp_ref[...]) * dinv
    lin = _mm(y, w2_ref[...], 1, 1) - _mm(y, w2_ref[...], 1, 0) - GAMMA * y
    z = lin + gcn + b2_ref[...]
    x2 = jnp.maximum(y + EPS * jnp.tanh(z), 0.0)
    x2_ref[...] = x2
    logits = _mm(x2, l2w_ref[...], 1, 1) + l2b_ref[...]
    m = jnp.max(logits, axis=-1, keepdims=True)
    lse = m + jnp.log(jnp.sum(jnp.exp(logits - m), axis=-1, keepdims=True))
    res_ref[...] = logits - lse


_t3 = pl.pallas_call(
    _t3_body,
    grid=(GRID,),
    in_specs=[
        pl.BlockSpec((BN, D), lambda i: (i, 0)),
        pl.BlockSpec((NC, BN, D), lambda i: (0, i, 0)),
        pl.BlockSpec((BN, D), lambda i: (i, 0)),
        pl.BlockSpec((BN, 1), lambda i: (i, 0)),
        pl.BlockSpec((D, D), lambda i: (0, 0)),
        pl.BlockSpec((1, D), lambda i: (0, 0)),
        pl.BlockSpec((C, D), lambda i: (0, 0)),
        pl.BlockSpec((1, C), lambda i: (0, 0)),
    ],
    out_specs=[
        pl.BlockSpec((BN, C), lambda i: (i, 0)),
        pl.BlockSpec((BN, D), lambda i: (i, 0)),
    ],
    out_shape=[
        jax.ShapeDtypeStruct((N, C), jnp.float32),
        jax.ShapeDtypeStruct((N, D), jnp.float32),
    ],
)


def kernel(x, edge_index, W1, b1, gcn_W1, l1_w, l1_b, W2, b2, gcn_W2,
           l2_w, l2_b):
    src = edge_index[0]
    dst = edge_index[1]
    zeros_rows = jnp.zeros((RPT, D), jnp.float32)
    ones_rows = jnp.ones((KE, D), jnp.float32)

    deg_parts = _sc_deg(dst, ones_rows, zeros_rows)
    dinv, h1p = _t1(deg_parts, x, gcn_W1)
    acc1 = _sc_scatter(h1p, src, dst, zeros_rows)
    y, h2p = _t2(x, acc1, h1p, dinv, W1, b1.reshape(1, D),
                 l1_w, l1_b.reshape(1, D), gcn_W2)
    acc2 = _sc_scatter(h2p, src, dst, zeros_rows)
    res, x2 = _t3(y, acc2, h2p, dinv, W2, b2.reshape(1, D),
                  l2_w, l2_b.reshape(1, C))
    return (res, x2)
